# baseline (speedup 1.0000x reference)
import functools

import jax
import jax.numpy as jnp
from jax import lax
from jax.experimental import pallas as pl
from jax.experimental.pallas import tpu as pltpu
from jax.experimental.pallas import tpu_sc as plsc

N = 10000
E = 320000
D = 128
DH = D // 2
DE = 16
L = 5

NC = 2
NS = 16
EPS = E // NS
C = 80
NCH = EPS // C
SB = 50
NSB = NCH // SB
RPS = 624
TAIL = N - NS * RPS
LANES = 16


def _vector_mesh():
    return plsc.VectorSubcoreMesh(core_axis_name="c", subcore_axis_name="s")


def _sc_layer(hs, src3, dst3, ep, zrows):
    @functools.partial(
        pl.kernel,
        out_type=jax.ShapeDtypeStruct((NC, N, DH), jnp.float32),
        mesh=_vector_mesh(),
        compiler_params=pltpu.CompilerParams(use_tc_tiling_on_sc=False),
        scratch_types=[
            pltpu.VMEM((SB, C), jnp.int32),
            pltpu.VMEM((SB, C), jnp.int32),
            pltpu.VMEM((C, DH), jnp.float32),
            pltpu.VMEM((C, DH), jnp.float32),
            pltpu.VMEM((C // 2, D), jnp.float32),
            pltpu.VMEM((C // 2, D), jnp.float32),
            pltpu.VMEM((C, DH), jnp.float32),
            pltpu.VMEM((C, DH), jnp.float32),
            pltpu.VMEM_SHARED((N, DH), jnp.float32),
            pltpu.SemaphoreType.DMA,
            pltpu.SemaphoreType.DMA,
            pltpu.SemaphoreType.DMA,
            pltpu.SemaphoreType.DMA,
        ],
    )
    def k(hs_hbm, src_hbm, dst_hbm, ep_hbm, z_hbm, out_hbm,
          src_sb, dst_sb, g0, g1, e0, e1, m0, m1, agg_sh,
          semL0, semL1, semS0, semS1):
        c = lax.axis_index("c")
        s = lax.axis_index("s")
        ebase = s * EPS
        gbufs = (g0, g1)
        ebufs = (e0, e1)
        mbufs = (m0, m1)
        semL = (semL0, semL1)
        semS = (semS0, semS1)

        pltpu.sync_copy(z_hbm.at[pl.ds(s * RPS, RPS)],
                        agg_sh.at[pl.ds(s * RPS, RPS)])

        @pl.when(s == NS - 1)
        def _():
            pltpu.sync_copy(z_hbm.at[pl.ds(NS * RPS, TAIL)],
                            agg_sh.at[pl.ds(NS * RPS, TAIL)])

        plsc.subcore_barrier()

        def issue(k_row, ch, p):
            pltpu.async_copy(hs_hbm.at[c].at[src_sb.at[k_row]],
                             gbufs[p], semL[p])
            pltpu.async_copy(
                ep_hbm.at[c, pl.ds(s * (EPS // 2) + ch * (C // 2), C // 2)],
                ebufs[p], semL[p])

        def wait_loads(k_row, ch, p):
            pltpu.make_async_copy(hs_hbm.at[c].at[src_sb.at[k_row]],
                                  gbufs[p], semL[p]).wait()
            pltpu.make_async_copy(
                ep_hbm.at[c, pl.ds(s * (EPS // 2) + ch * (C // 2), C // 2)],
                ebufs[p], semL[p]).wait()

        def compute(p):
            g_buf, e_buf, m_buf = gbufs[p], ebufs[p], mbufs[p]

            @pl.loop(0, C // 2, step=4)
            def _(hp):
                for dp in range(4):
                    for par in range(2):
                        r_off = 2 * dp + par
                        for cc in range(DH // LANES):
                            sl = pl.ds(cc * LANES, LANES)
                            esl = pl.ds(par * DH + cc * LANES, LANES)
                            m_buf[2 * hp + r_off, sl] = jnp.maximum(
                                g_buf[2 * hp + r_off, sl]
                                + e_buf[hp + dp, esl], 0.0)

        def issue_scatter(k_row, p):
            pltpu.async_copy(mbufs[p], agg_sh.at[dst_sb.at[k_row]],
                             semS[p], add=True)

        def wait_scatter(k_row, p):
            pltpu.make_async_copy(mbufs[p], agg_sh.at[dst_sb.at[k_row]],
                                  semS[p]).wait()

        @pl.loop(0, NSB)
        def _(t):
            cb = t * SB
            pltpu.sync_copy(src_hbm.at[s, pl.ds(cb, SB)], src_sb)
            pltpu.sync_copy(dst_hbm.at[s, pl.ds(cb, SB)], dst_sb)
            issue(0, cb, 0)

            @pl.loop(0, SB // 2)
            def _(j):
                k0 = 2 * j
                issue(k0 + 1, cb + k0 + 1, 1)
                wait_loads(k0, cb + k0, 0)

                @pl.when(j > 0)
                def _():
                    wait_scatter(k0 - 2, 0)

                compute(0)
                issue_scatter(k0, 0)

                @pl.when(j < SB // 2 - 1)
                def _():
                    issue(k0 + 2, cb + k0 + 2, 0)

                wait_loads(k0 + 1, cb + k0 + 1, 1)

                @pl.when(j > 0)
                def _():
                    wait_scatter(k0 - 1, 1)

                compute(1)
                issue_scatter(k0 + 1, 1)

            wait_scatter(SB - 2, 0)
            wait_scatter(SB - 1, 1)

        plsc.subcore_barrier()
        pltpu.sync_copy(agg_sh.at[pl.ds(s * RPS, RPS)],
                        out_hbm.at[c, pl.ds(s * RPS, RPS)])

        @pl.when(s == NS - 1)
        def _():
            pltpu.sync_copy(agg_sh.at[pl.ds(NS * RPS, TAIL)],
                            out_hbm.at[c, pl.ds(NS * RPS, TAIL)])

    return k(hs, src3, dst3, ep, zrows)


_EB = 2000


def _tc_eproj(ef, We, be):
    def body(ef_ref, we_ref, be_ref, out_ref):
        y = jnp.dot(ef_ref[...], we_ref[...],
                    preferred_element_type=jnp.float32) + be_ref[...]
        z = y.reshape(_EB, 2, D)
        for cidx in range(NC):
            sl = slice(cidx * DH, (cidx + 1) * DH)
            out_ref[cidx] = jnp.concatenate([z[:, 0, sl], z[:, 1, sl]],
                                            axis=1)

    return pl.pallas_call(
        body,
        grid=(E // 2 // _EB,),
        in_specs=[
            pl.BlockSpec((2 * _EB, DE), lambda i: (i, 0)),
            pl.BlockSpec((DE, D), lambda i: (0, 0)),
            pl.BlockSpec((1, D), lambda i: (0, 0)),
        ],
        out_specs=pl.BlockSpec((NC, _EB, D), lambda i: (0, i, 0)),
        out_shape=jax.ShapeDtypeStruct((NC, E // 2, D), jnp.float32),
    )(ef, We, be)


_NB = 2000


def _tc_update(hs, agg, W, b):
    def body(h_ref, p_ref, w_ref, b_ref, os_ref, of_ref):
        t = jnp.concatenate(
            [h_ref[0] + p_ref[0], h_ref[1] + p_ref[1]], axis=1)
        y = jnp.dot(t, w_ref[...], preferred_element_type=jnp.float32) \
            + b_ref[...]
        z = jnp.where(y > 0.0, y, jnp.exp(jnp.minimum(y, 0.0)) - 1.0)
        os_ref[0] = z[:, :DH]
        os_ref[1] = z[:, DH:]
        of_ref[...] = z

    return pl.pallas_call(
        body,
        grid=(N // _NB,),
        in_specs=[
            pl.BlockSpec((NC, _NB, DH), lambda i: (0, i, 0)),
            pl.BlockSpec((NC, _NB, DH), lambda i: (0, i, 0)),
            pl.BlockSpec((D, D), lambda i: (0, 0)),
            pl.BlockSpec((1, D), lambda i: (0, 0)),
        ],
        out_specs=[
            pl.BlockSpec((NC, _NB, DH), lambda i: (0, i, 0)),
            pl.BlockSpec((_NB, D), lambda i: (i, 0)),
        ],
        out_shape=[
            jax.ShapeDtypeStruct((NC, N, DH), jnp.float32),
            jax.ShapeDtypeStruct((N, D), jnp.float32),
        ],
    )(hs, agg, W, b)


def kernel(x, edge_index, edge_feats, Ws, bs, Wes, bes):
    src3 = edge_index[0].reshape(NS, NCH, C)
    dst3 = edge_index[1].reshape(NS, NCH, C)
    zrows = jnp.zeros((N, DH), jnp.float32)
    eps = [_tc_eproj(edge_feats, Wes[i], bes[i].reshape(1, D))
           for i in range(L)]
    hs = jnp.stack([x[:, :DH], x[:, DH:]])
    hf = x
    for i in range(L):
        agg = _sc_layer(hs, src3, dst3, eps[i], zrows)
        hs, hf = _tc_update(hs, agg, Ws[i], bs[i].reshape(1, D))
    return hf


# speedup vs baseline: 1.0049x; 1.0049x over previous
import functools

import jax
import jax.numpy as jnp
from jax import lax
from jax.experimental import pallas as pl
from jax.experimental.pallas import tpu as pltpu
from jax.experimental.pallas import tpu_sc as plsc

N = 10000
E = 320000
D = 128
DH = D // 2
DE = 16
L = 5

NC = 2
NS = 16
EPS = E // NS
C = 80
NCH = EPS // C
SB = 50
NSB = NCH // SB
RPS = 624
TAIL = N - NS * RPS
LANES = 16


def _vector_mesh():
    return plsc.VectorSubcoreMesh(core_axis_name="c", subcore_axis_name="s")


def _sc_layer(hs, src3, dst3, ep, zrows):
    @functools.partial(
        pl.kernel,
        out_type=jax.ShapeDtypeStruct((NC, N, DH), jnp.float32),
        mesh=_vector_mesh(),
        compiler_params=pltpu.CompilerParams(use_tc_tiling_on_sc=False),
        scratch_types=[
            pltpu.VMEM((SB, C), jnp.int32),
            pltpu.VMEM((SB, C), jnp.int32),
            pltpu.VMEM((C, DH), jnp.float32),
            pltpu.VMEM((C, DH), jnp.float32),
            pltpu.VMEM((C // 2, D), jnp.float32),
            pltpu.VMEM((C // 2, D), jnp.float32),
            pltpu.VMEM((C, DH), jnp.float32),
            pltpu.VMEM((C, DH), jnp.float32),
            pltpu.VMEM_SHARED((N, DH), jnp.float32),
            pltpu.SemaphoreType.DMA,
            pltpu.SemaphoreType.DMA,
            pltpu.SemaphoreType.DMA,
            pltpu.SemaphoreType.DMA,
        ],
    )
    def k(hs_hbm, src_hbm, dst_hbm, ep_hbm, z_hbm, out_hbm,
          src_sb, dst_sb, g0, g1, e0, e1, m0, m1, agg_sh,
          semL0, semL1, semS0, semS1):
        c = lax.axis_index("c")
        s = lax.axis_index("s")
        ebase = s * EPS
        gbufs = (g0, g1)
        ebufs = (e0, e1)
        mbufs = (m0, m1)
        semL = (semL0, semL1)
        semS = (semS0, semS1)

        pltpu.sync_copy(z_hbm.at[pl.ds(s * RPS, RPS)],
                        agg_sh.at[pl.ds(s * RPS, RPS)])

        @pl.when(s == NS - 1)
        def _():
            pltpu.sync_copy(z_hbm.at[pl.ds(NS * RPS, TAIL)],
                            agg_sh.at[pl.ds(NS * RPS, TAIL)])

        plsc.subcore_barrier()

        def issue(k_row, ch, p):
            pltpu.async_copy(hs_hbm.at[c].at[src_sb.at[k_row]],
                             gbufs[p], semL[p])
            pltpu.async_copy(
                ep_hbm.at[c, pl.ds(s * (EPS // 2) + ch * (C // 2), C // 2)],
                ebufs[p], semL[p])

        def wait_loads(k_row, ch, p):
            pltpu.make_async_copy(hs_hbm.at[c].at[src_sb.at[k_row]],
                                  gbufs[p], semL[p]).wait()
            pltpu.make_async_copy(
                ep_hbm.at[c, pl.ds(s * (EPS // 2) + ch * (C // 2), C // 2)],
                ebufs[p], semL[p]).wait()

        def compute(p):
            g_buf, e_buf, m_buf = gbufs[p], ebufs[p], mbufs[p]

            @pl.loop(0, C // 2, step=4)
            def _(hp):
                for dp in range(4):
                    for par in range(2):
                        r_off = 2 * dp + par
                        for cc in range(DH // LANES):
                            sl = pl.ds(cc * LANES, LANES)
                            esl = pl.ds(par * DH + cc * LANES, LANES)
                            m_buf[2 * hp + r_off, sl] = jnp.maximum(
                                g_buf[2 * hp + r_off, sl]
                                + e_buf[hp + dp, esl], 0.0)

        def issue_scatter(k_row, p):
            pltpu.async_copy(mbufs[p], agg_sh.at[dst_sb.at[k_row]],
                             semS[p], add=True)

        def wait_scatter(k_row, p):
            pltpu.make_async_copy(mbufs[p], agg_sh.at[dst_sb.at[k_row]],
                                  semS[p]).wait()

        @pl.loop(0, NSB)
        def _(t):
            cb = t * SB
            pltpu.sync_copy(src_hbm.at[s, pl.ds(cb, SB)], src_sb)
            pltpu.sync_copy(dst_hbm.at[s, pl.ds(cb, SB)], dst_sb)
            issue(0, cb, 0)

            @pl.loop(0, SB // 2)
            def _(j):
                k0 = 2 * j
                issue(k0 + 1, cb + k0 + 1, 1)
                wait_loads(k0, cb + k0, 0)

                @pl.when(j > 0)
                def _():
                    wait_scatter(k0 - 2, 0)

                compute(0)
                issue_scatter(k0, 0)

                @pl.when(j < SB // 2 - 1)
                def _():
                    issue(k0 + 2, cb + k0 + 2, 0)

                wait_loads(k0 + 1, cb + k0 + 1, 1)

                @pl.when(j > 0)
                def _():
                    wait_scatter(k0 - 1, 1)

                compute(1)
                issue_scatter(k0 + 1, 1)

            wait_scatter(SB - 2, 0)
            wait_scatter(SB - 1, 1)

        plsc.subcore_barrier()
        pltpu.sync_copy(agg_sh.at[pl.ds(s * RPS, RPS)],
                        out_hbm.at[c, pl.ds(s * RPS, RPS)])

        @pl.when(s == NS - 1)
        def _():
            pltpu.sync_copy(agg_sh.at[pl.ds(NS * RPS, TAIL)],
                            out_hbm.at[c, pl.ds(NS * RPS, TAIL)])

    return k(hs, src3, dst3, ep, zrows)


_EB = 2000


def _tc_eproj(ef, We, be):
    zde = jnp.zeros((DE, DH), jnp.float32)
    W2 = jnp.stack([
        jnp.concatenate([
            jnp.concatenate([We[:, cidx * DH:(cidx + 1) * DH], zde], axis=1),
            jnp.concatenate([zde, We[:, cidx * DH:(cidx + 1) * DH]], axis=1),
        ], axis=0)
        for cidx in range(NC)])
    b2 = jnp.stack([
        jnp.concatenate([be[0, cidx * DH:(cidx + 1) * DH]] * 2).reshape(1, D)
        for cidx in range(NC)])

    def body(ef_ref, w_ref, b_ref, out_ref):
        z = ef_ref[...].reshape(_EB, 2, DE)
        ef2 = jnp.concatenate([z[:, 0], z[:, 1]], axis=1)
        for cidx in range(NC):
            out_ref[cidx] = jnp.dot(
                ef2, w_ref[cidx],
                preferred_element_type=jnp.float32) + b_ref[cidx]

    return pl.pallas_call(
        body,
        grid=(E // 2 // _EB,),
        in_specs=[
            pl.BlockSpec((2 * _EB, DE), lambda i: (i, 0)),
            pl.BlockSpec((NC, 2 * DE, D), lambda i: (0, 0, 0)),
            pl.BlockSpec((NC, 1, D), lambda i: (0, 0, 0)),
        ],
        out_specs=pl.BlockSpec((NC, _EB, D), lambda i: (0, i, 0)),
        out_shape=jax.ShapeDtypeStruct((NC, E // 2, D), jnp.float32),
    )(ef, W2, b2)


_NB = 2000


def _tc_update(hs, agg, W, b):
    def body(h_ref, p_ref, w_ref, b_ref, os_ref, of_ref):
        t = jnp.concatenate(
            [h_ref[0] + p_ref[0], h_ref[1] + p_ref[1]], axis=1)
        y = jnp.dot(t, w_ref[...], preferred_element_type=jnp.float32) \
            + b_ref[...]
        z = jnp.where(y > 0.0, y, jnp.exp(jnp.minimum(y, 0.0)) - 1.0)
        os_ref[0] = z[:, :DH]
        os_ref[1] = z[:, DH:]
        of_ref[...] = z

    return pl.pallas_call(
        body,
        grid=(N // _NB,),
        in_specs=[
            pl.BlockSpec((NC, _NB, DH), lambda i: (0, i, 0)),
            pl.BlockSpec((NC, _NB, DH), lambda i: (0, i, 0)),
            pl.BlockSpec((D, D), lambda i: (0, 0)),
            pl.BlockSpec((1, D), lambda i: (0, 0)),
        ],
        out_specs=[
            pl.BlockSpec((NC, _NB, DH), lambda i: (0, i, 0)),
            pl.BlockSpec((_NB, D), lambda i: (i, 0)),
        ],
        out_shape=[
            jax.ShapeDtypeStruct((NC, N, DH), jnp.float32),
            jax.ShapeDtypeStruct((N, D), jnp.float32),
        ],
    )(hs, agg, W, b)


def kernel(x, edge_index, edge_feats, Ws, bs, Wes, bes):
    src3 = edge_index[0].reshape(NS, NCH, C)
    dst3 = edge_index[1].reshape(NS, NCH, C)
    zrows = jnp.zeros((N, DH), jnp.float32)
    eps = [_tc_eproj(edge_feats, Wes[i], bes[i].reshape(1, D))
           for i in range(L)]
    hs = jnp.stack([x[:, :DH], x[:, DH:]])
    hf = x
    for i in range(L):
        agg = _sc_layer(hs, src3, dst3, eps[i], zrows)
        hs, hf = _tc_update(hs, agg, Ws[i], bs[i].reshape(1, D))
    return hf


# speedup vs baseline: 1.0380x; 1.0329x over previous
import functools

import jax
import jax.numpy as jnp
from jax import lax
from jax.experimental import pallas as pl
from jax.experimental.pallas import tpu as pltpu
from jax.experimental.pallas import tpu_sc as plsc

N = 10000
E = 320000
D = 128
DH = D // 2
DE = 16
L = 5

NC = 2
NS = 16
EPS = E // NS
C = 80
NCH = EPS // C
SB = 50
NSB = NCH // SB
RPS = 624
TAIL = N - NS * RPS
LANES = 16


def _vector_mesh():
    return plsc.VectorSubcoreMesh(core_axis_name="c", subcore_axis_name="s")


def _sc_layer(hs, src3, dst3, ep, zrows, packed=True):
    e_shape = (C // 2, D) if packed else (C, DH)

    @functools.partial(
        pl.kernel,
        out_type=jax.ShapeDtypeStruct((NC, N, DH), jnp.float32),
        mesh=_vector_mesh(),
        compiler_params=pltpu.CompilerParams(use_tc_tiling_on_sc=False),
        scratch_types=[
            pltpu.VMEM((SB, C), jnp.int32),
            pltpu.VMEM((SB, C), jnp.int32),
            pltpu.VMEM((C, DH), jnp.float32),
            pltpu.VMEM((C, DH), jnp.float32),
            pltpu.VMEM(e_shape, jnp.float32),
            pltpu.VMEM(e_shape, jnp.float32),
            pltpu.VMEM((C, DH), jnp.float32),
            pltpu.VMEM((C, DH), jnp.float32),
            pltpu.VMEM_SHARED((N, DH), jnp.float32),
            pltpu.SemaphoreType.DMA,
            pltpu.SemaphoreType.DMA,
            pltpu.SemaphoreType.DMA,
            pltpu.SemaphoreType.DMA,
        ],
    )
    def k(hs_hbm, src_hbm, dst_hbm, ep_hbm, z_hbm, out_hbm,
          src_sb, dst_sb, g0, g1, e0, e1, m0, m1, agg_sh,
          semL0, semL1, semS0, semS1):
        c = lax.axis_index("c")
        s = lax.axis_index("s")
        ebase = s * EPS
        gbufs = (g0, g1)
        ebufs = (e0, e1)
        mbufs = (m0, m1)
        semL = (semL0, semL1)
        semS = (semS0, semS1)

        pltpu.sync_copy(z_hbm.at[pl.ds(s * RPS, RPS)],
                        agg_sh.at[pl.ds(s * RPS, RPS)])

        @pl.when(s == NS - 1)
        def _():
            pltpu.sync_copy(z_hbm.at[pl.ds(NS * RPS, TAIL)],
                            agg_sh.at[pl.ds(NS * RPS, TAIL)])

        plsc.subcore_barrier()

        def e_slice(ch):
            if packed:
                return ep_hbm.at[
                    c, pl.ds(s * (EPS // 2) + ch * (C // 2), C // 2)]
            return ep_hbm.at[pl.ds(s * EPS + ch * C, C), pl.ds(c * DH, DH)]

        def issue(k_row, ch, p):
            pltpu.async_copy(hs_hbm.at[c].at[src_sb.at[k_row]],
                             gbufs[p], semL[p])
            pltpu.async_copy(e_slice(ch), ebufs[p], semL[p])

        def wait_loads(k_row, ch, p):
            pltpu.make_async_copy(hs_hbm.at[c].at[src_sb.at[k_row]],
                                  gbufs[p], semL[p]).wait()
            pltpu.make_async_copy(e_slice(ch), ebufs[p], semL[p]).wait()

        def compute(p):
            g_buf, e_buf, m_buf = gbufs[p], ebufs[p], mbufs[p]

            if packed:
                @pl.loop(0, C // 2, step=4)
                def _(hp):
                    for dp in range(4):
                        for par in range(2):
                            r_off = 2 * dp + par
                            for cc in range(DH // LANES):
                                sl = pl.ds(cc * LANES, LANES)
                                esl = pl.ds(par * DH + cc * LANES, LANES)
                                m_buf[2 * hp + r_off, sl] = jnp.maximum(
                                    g_buf[2 * hp + r_off, sl]
                                    + e_buf[hp + dp, esl], 0.0)
            else:
                @pl.loop(0, C, step=8)
                def _(r0):
                    for dr in range(8):
                        for cc in range(DH // LANES):
                            sl = pl.ds(cc * LANES, LANES)
                            m_buf[r0 + dr, sl] = jnp.maximum(
                                g_buf[r0 + dr, sl] + e_buf[r0 + dr, sl], 0.0)

        def issue_scatter(k_row, p):
            pltpu.async_copy(mbufs[p], agg_sh.at[dst_sb.at[k_row]],
                             semS[p], add=True)

        def wait_scatter(k_row, p):
            pltpu.make_async_copy(mbufs[p], agg_sh.at[dst_sb.at[k_row]],
                                  semS[p]).wait()

        @pl.loop(0, NSB)
        def _(t):
            cb = t * SB
            pltpu.sync_copy(src_hbm.at[s, pl.ds(cb, SB)], src_sb)
            pltpu.sync_copy(dst_hbm.at[s, pl.ds(cb, SB)], dst_sb)
            issue(0, cb, 0)

            @pl.loop(0, SB // 2)
            def _(j):
                k0 = 2 * j
                issue(k0 + 1, cb + k0 + 1, 1)
                wait_loads(k0, cb + k0, 0)

                @pl.when(j > 0)
                def _():
                    wait_scatter(k0 - 2, 0)

                compute(0)
                issue_scatter(k0, 0)

                @pl.when(j < SB // 2 - 1)
                def _():
                    issue(k0 + 2, cb + k0 + 2, 0)

                wait_loads(k0 + 1, cb + k0 + 1, 1)

                @pl.when(j > 0)
                def _():
                    wait_scatter(k0 - 1, 1)

                compute(1)
                issue_scatter(k0 + 1, 1)

            wait_scatter(SB - 2, 0)
            wait_scatter(SB - 1, 1)

        plsc.subcore_barrier()
        pltpu.sync_copy(agg_sh.at[pl.ds(s * RPS, RPS)],
                        out_hbm.at[c, pl.ds(s * RPS, RPS)])

        @pl.when(s == NS - 1)
        def _():
            pltpu.sync_copy(agg_sh.at[pl.ds(NS * RPS, TAIL)],
                            out_hbm.at[c, pl.ds(NS * RPS, TAIL)])

    return k(hs, src3, dst3, ep, zrows)


_EB = 2000


def _tc_eproj_packed(ef2, We, be):
    zde = jnp.zeros((DE, DH), jnp.float32)
    W2 = jnp.stack([
        jnp.concatenate([
            jnp.concatenate([We[:, cidx * DH:(cidx + 1) * DH], zde], axis=1),
            jnp.concatenate([zde, We[:, cidx * DH:(cidx + 1) * DH]], axis=1),
        ], axis=0)
        for cidx in range(NC)])
    b2 = jnp.stack([
        jnp.concatenate([be[0, cidx * DH:(cidx + 1) * DH]] * 2).reshape(1, D)
        for cidx in range(NC)])

    def body(ef_ref, w_ref, b_ref, out_ref):
        for cidx in range(NC):
            out_ref[cidx] = jnp.dot(
                ef_ref[...], w_ref[cidx],
                preferred_element_type=jnp.float32) + b_ref[cidx]

    return pl.pallas_call(
        body,
        grid=(E // 2 // _EB,),
        in_specs=[
            pl.BlockSpec((_EB, 2 * DE), lambda i: (i, 0)),
            pl.BlockSpec((NC, 2 * DE, D), lambda i: (0, 0, 0)),
            pl.BlockSpec((NC, 1, D), lambda i: (0, 0, 0)),
        ],
        out_specs=pl.BlockSpec((NC, _EB, D), lambda i: (0, i, 0)),
        out_shape=jax.ShapeDtypeStruct((NC, E // 2, D), jnp.float32),
    )(ef2, W2, b2)


def _tc_eproj_plain(ef, We, be):
    def body(ef_ref, we_ref, be_ref, out_ref):
        out_ref[...] = jnp.dot(ef_ref[...], we_ref[...],
                               preferred_element_type=jnp.float32) + be_ref[...]

    return pl.pallas_call(
        body,
        grid=(E // (2 * _EB),),
        in_specs=[
            pl.BlockSpec((2 * _EB, DE), lambda i: (i, 0)),
            pl.BlockSpec((DE, D), lambda i: (0, 0)),
            pl.BlockSpec((1, D), lambda i: (0, 0)),
        ],
        out_specs=pl.BlockSpec((2 * _EB, D), lambda i: (i, 0)),
        out_shape=jax.ShapeDtypeStruct((E, D), jnp.float32),
    )(ef, We, be)


_NB = 2000


def _tc_update(hs, agg, W, b):
    def body(h_ref, p_ref, w_ref, b_ref, os_ref, of_ref):
        t = jnp.concatenate(
            [h_ref[0] + p_ref[0], h_ref[1] + p_ref[1]], axis=1)
        y = jnp.dot(t, w_ref[...], preferred_element_type=jnp.float32) \
            + b_ref[...]
        z = jnp.where(y > 0.0, y, jnp.exp(jnp.minimum(y, 0.0)) - 1.0)
        os_ref[0] = z[:, :DH]
        os_ref[1] = z[:, DH:]
        of_ref[...] = z

    return pl.pallas_call(
        body,
        grid=(N // _NB,),
        in_specs=[
            pl.BlockSpec((NC, _NB, DH), lambda i: (0, i, 0)),
            pl.BlockSpec((NC, _NB, DH), lambda i: (0, i, 0)),
            pl.BlockSpec((D, D), lambda i: (0, 0)),
            pl.BlockSpec((1, D), lambda i: (0, 0)),
        ],
        out_specs=[
            pl.BlockSpec((NC, _NB, DH), lambda i: (0, i, 0)),
            pl.BlockSpec((_NB, D), lambda i: (i, 0)),
        ],
        out_shape=[
            jax.ShapeDtypeStruct((NC, N, DH), jnp.float32),
            jax.ShapeDtypeStruct((N, D), jnp.float32),
        ],
    )(hs, agg, W, b)


def kernel(x, edge_index, edge_feats, Ws, bs, Wes, bes):
    src3 = edge_index[0].reshape(NS, NCH, C)
    dst3 = edge_index[1].reshape(NS, NCH, C)
    zrows = jnp.zeros((N, DH), jnp.float32)
    ef2 = edge_feats.reshape(E // 2, 2 * DE)
    eps = [_tc_eproj_plain(edge_feats, Wes[0], bes[0].reshape(1, D))]
    eps += [_tc_eproj_packed(ef2, Wes[i], bes[i].reshape(1, D))
            for i in range(1, L)]
    hs = jnp.stack([x[:, :DH], x[:, DH:]])
    hf = x
    for i in range(L):
        agg = _sc_layer(hs, src3, dst3, eps[i], zrows, packed=(i > 0))
        hs, hf = _tc_update(hs, agg, Ws[i], bs[i].reshape(1, D))
    return hf
